# baseline (speedup 1.0000x reference)
import functools

import jax
import jax.numpy as jnp
from jax import lax
from jax.experimental import pallas as pl
from jax.experimental.pallas import tpu as pltpu
from jax.experimental.pallas import tpu_sc as plsc

_T = 4096
_D = 2048
_E = 64
_C = 256
_L = 16
_TB = 512
_ZROW = 2 * _T


def _routing_body(logits_ref, src_ref, w1_ref, w2_ref, probs_ref, dstx_ref):
    e = pl.program_id(0)

    @pl.when(e == 0)
    def _init():
        l = logits_ref[...]
        m = jnp.max(l, axis=1, keepdims=True)
        p = jnp.exp(l - m)
        probs = p / jnp.sum(p, axis=1, keepdims=True)
        probs_ref[...] = probs
        cols = lax.broadcasted_iota(jnp.int32, (_T, _E), 1)
        m1 = jnp.max(probs, axis=1, keepdims=True)
        i1 = jnp.min(jnp.where(probs == m1, cols, _E), axis=1, keepdims=True)
        p2 = jnp.where(cols == i1, -jnp.inf, probs)
        m2 = jnp.max(p2, axis=1, keepdims=True)
        i2 = jnp.min(jnp.where(p2 == m2, cols, _E), axis=1, keepdims=True)
        is2 = (cols == i2).astype(jnp.int32)
        mask = (cols == i1).astype(jnp.int32) + is2
        c = mask
        k = 1
        while k < _T:
            c = c + jnp.concatenate(
                [jnp.zeros((k, _E), jnp.int32), c[:-k]], axis=0)
            k *= 2
        dstx_ref[...] = c * mask + 16384 * is2
        w1_ref[...] = jnp.broadcast_to(m1, (_T, 128))
        w2_ref[...] = jnp.broadcast_to(m2, (_T, 128))

    e_cols = lax.broadcasted_iota(jnp.int32, (_T, _E), 1)
    sel = e_cols == e
    dstx_col = jnp.sum(jnp.where(sel, dstx_ref[...], 0), axis=1)
    c2 = dstx_col // 16384
    dval = dstx_col - c2 * 16384
    slot = lax.broadcasted_iota(jnp.int32, (_C, _T), 0) + 1
    eq = slot == dval[None, :]
    ids = lax.broadcasted_iota(jnp.int32, (_C, _T), 1)
    idsx = ids + _T * c2[None, :]
    src_row = jnp.sum(jnp.where(eq, idsx - _ZROW, 0), axis=1) + _ZROW
    src_ref[...] = src_row.reshape(1, 1, _C)


def _routing(logits):
    return pl.pallas_call(
        _routing_body,
        grid=(_E,),
        in_specs=[pl.BlockSpec((_T, _E), lambda e: (0, 0))],
        out_specs=[
            pl.BlockSpec((1, 1, _C), lambda e: (e, 0, 0)),
            pl.BlockSpec((_T, 128), lambda e: (0, 0)),
            pl.BlockSpec((_T, 128), lambda e: (0, 0)),
        ],
        out_shape=[
            jax.ShapeDtypeStruct((_E, 1, _C), jnp.int32),
            jax.ShapeDtypeStruct((_T, 128), jnp.float32),
            jax.ShapeDtypeStruct((_T, 128), jnp.float32),
        ],
        scratch_shapes=[
            pltpu.VMEM((_T, _E), jnp.float32),
            pltpu.VMEM((_T, _E), jnp.int32),
        ],
    )(logits)


def _tables_body(x_ref, w1_ref, w2_ref, out_ref):
    s = pl.program_id(0)
    xb = x_ref[...]
    w1c = w1_ref[...][:, 0:1]
    w2c = w2_ref[...][:, 0:1]
    wc = jnp.where(s < 8, w1c, w2c)
    wc = jnp.where(s < 16, wc, 0.0)
    out_ref[...] = xb * wc


def _tables(x, w1p, w2p):
    nblk = _T // _TB
    nsteps = 2 * nblk + 1
    return pl.pallas_call(
        _tables_body,
        grid=(nsteps,),
        in_specs=[
            pl.BlockSpec((_TB, _D), lambda s: (s % 8, 0)),
            pl.BlockSpec((_TB, 128), lambda s: (s % 8, 0)),
            pl.BlockSpec((_TB, 128), lambda s: (s % 8, 0)),
        ],
        out_specs=pl.BlockSpec((_TB, _D), lambda s: (s, 0)),
        out_shape=jax.ShapeDtypeStruct((nsteps * _TB, _D), jnp.float32),
    )(x, w1p, w2p)


_NBUF = 6
_RCH = 8
_LEAD = 3


def _dispatch(xs, src_flat):
    info = plsc.get_sparse_core_info()
    nc, ns = info.num_cores, info.num_subcores
    nw = nc * ns
    b = _E * _C
    bpw = b // nw
    r = _RCH
    nch = bpw // r

    sl = _D // 128

    @functools.partial(
        pl.kernel,
        mesh=plsc.VectorSubcoreMesh(core_axis_name="c", subcore_axis_name="s"),
        compiler_params=pltpu.CompilerParams(use_tc_tiling_on_sc=True),
        out_type=jax.ShapeDtypeStruct((b, sl, 128), jnp.float32),
        scratch_types=(
            [pltpu.VMEM((bpw,), jnp.int32)]
            + [pltpu.VMEM((r, sl, 128), jnp.float32)] * _NBUF
            + [pltpu.SemaphoreType.DMA] * (2 * _NBUF)
        ),
    )
    def k(xs_hbm, src_hbm, out_hbm, idx_v, *bufsem):
        bufs = bufsem[:_NBUF]
        gsem = bufsem[_NBUF:2 * _NBUF]
        ssem = bufsem[2 * _NBUF:]
        wid = lax.axis_index("s") * nc + lax.axis_index("c")
        base = wid * bpw
        pltpu.sync_copy(src_hbm.at[pl.ds(base, bpw)], idx_v)

        def start_gather(g, bi):
            pltpu.async_copy(
                xs_hbm.at[idx_v.at[pl.ds(g * r, r)]], bufs[bi], gsem[bi])

        def wait_dma(bi, sem):
            pltpu.make_async_copy(
                xs_hbm.at[pl.ds(0, r)], bufs[bi], sem[bi]).wait()

        def start_store(g, bi):
            pltpu.async_copy(
                bufs[bi], out_hbm.at[pl.ds(base + g * r, r)], ssem[bi])

        for g in range(_LEAD):
            start_gather(g, g)

        def step(kk, carry):
            for bi in range(_NBUF):
                g = _NBUF * kk + bi

                @pl.when(g < nch)
                def _(g=g, bi=bi):
                    wait_dma(bi, gsem)
                    start_store(g, bi)
                    bn = (bi + _LEAD) % _NBUF

                    @pl.when(g + _LEAD < nch)
                    def _():
                        @pl.when(g >= _NBUF - _LEAD)
                        def _():
                            wait_dma(bn, ssem)
                        start_gather(g + _LEAD, bn)

            return carry

        lax.fori_loop(0, (nch + _NBUF - 1) // _NBUF, step, 0)
        for g in range(nch - _NBUF, nch):
            wait_dma(g % _NBUF, ssem)

    return k(xs.reshape(xs.shape[0], sl, 128), src_flat)


def kernel(x, router_logits):
    src3, w1p, w2p = _routing(router_logits)
    xs = _tables(x, w1p, w2p)
    out = _dispatch(xs, src3.reshape(_E * _C))
    return out.reshape(_E, _C, _D)


# speedup vs baseline: 1.0839x; 1.0839x over previous
import functools

import jax
import jax.numpy as jnp
from jax import lax
from jax.experimental import pallas as pl
from jax.experimental.pallas import tpu as pltpu
from jax.experimental.pallas import tpu_sc as plsc

_T = 4096
_D = 2048
_E = 64
_C = 256
_L = 16
_TB = 512
_ZROW = 2 * _T


def _routing_body(logits_ref, src_ref, w1_ref, w2_ref, probs_ref, dstx_ref):
    e = pl.program_id(0)

    @pl.when(e == 0)
    def _init():
        l = logits_ref[...]
        m = jnp.max(l, axis=1, keepdims=True)
        p = jnp.exp(l - m)
        probs = p / jnp.sum(p, axis=1, keepdims=True)
        probs_ref[...] = probs
        cols = lax.broadcasted_iota(jnp.int32, (_T, _E), 1)
        m1 = jnp.max(probs, axis=1, keepdims=True)
        i1 = jnp.min(jnp.where(probs == m1, cols, _E), axis=1, keepdims=True)
        p2 = jnp.where(cols == i1, -jnp.inf, probs)
        m2 = jnp.max(p2, axis=1, keepdims=True)
        i2 = jnp.min(jnp.where(p2 == m2, cols, _E), axis=1, keepdims=True)
        is2 = (cols == i2).astype(jnp.int32)
        mask = (cols == i1).astype(jnp.int32) + is2
        c = mask
        k = 1
        while k < _T:
            c = c + jnp.concatenate(
                [jnp.zeros((k, _E), jnp.int32), c[:-k]], axis=0)
            k *= 2
        dstx_ref[...] = c * mask + 16384 * is2
        w1_ref[...] = jnp.broadcast_to(m1, (_T, 128))
        w2_ref[...] = jnp.broadcast_to(m2, (_T, 128))

    e_cols = lax.broadcasted_iota(jnp.int32, (_T, _E), 1)
    sel = e_cols == e
    dstx_col = jnp.sum(jnp.where(sel, dstx_ref[...], 0), axis=1)
    c2 = dstx_col // 16384
    dval = dstx_col - c2 * 16384
    slot = lax.broadcasted_iota(jnp.int32, (_T, _C), 1) + 1
    eq = slot == dval[:, None]
    ids = lax.broadcasted_iota(jnp.int32, (_T, _C), 0)
    idsx = ids + _T * c2[:, None]
    src_row = jnp.sum(jnp.where(eq, idsx - _ZROW, 0), axis=0) + _ZROW
    src_ref[...] = src_row.reshape(1, 1, _C)


def _routing(logits):
    return pl.pallas_call(
        _routing_body,
        grid=(_E,),
        in_specs=[pl.BlockSpec((_T, _E), lambda e: (0, 0))],
        out_specs=[
            pl.BlockSpec((1, 1, _C), lambda e: (e, 0, 0)),
            pl.BlockSpec((_T, 128), lambda e: (0, 0)),
            pl.BlockSpec((_T, 128), lambda e: (0, 0)),
        ],
        out_shape=[
            jax.ShapeDtypeStruct((_E, 1, _C), jnp.int32),
            jax.ShapeDtypeStruct((_T, 128), jnp.float32),
            jax.ShapeDtypeStruct((_T, 128), jnp.float32),
        ],
        scratch_shapes=[
            pltpu.VMEM((_T, _E), jnp.float32),
            pltpu.VMEM((_T, _E), jnp.int32),
        ],
    )(logits)


def _tables_body(x_ref, w1_ref, w2_ref, out_ref):
    s = pl.program_id(0)
    xb = x_ref[...]
    w1c = w1_ref[...][:, 0:1]
    w2c = w2_ref[...][:, 0:1]
    wc = jnp.where(s < 8, w1c, w2c)
    wc = jnp.where(s < 16, wc, 0.0)
    out_ref[...] = (xb * wc).reshape(_TB, _D // 128, 128)


def _tables(x, w1p, w2p):
    nblk = _T // _TB
    nsteps = 2 * nblk + 1
    return pl.pallas_call(
        _tables_body,
        grid=(nsteps,),
        in_specs=[
            pl.BlockSpec((_TB, _D), lambda s: (s % 8, 0)),
            pl.BlockSpec((_TB, 128), lambda s: (s % 8, 0)),
            pl.BlockSpec((_TB, 128), lambda s: (s % 8, 0)),
        ],
        out_specs=pl.BlockSpec((_TB, _D // 128, 128), lambda s: (s, 0, 0)),
        out_shape=jax.ShapeDtypeStruct(
            (nsteps * _TB, _D // 128, 128), jnp.float32),
    )(x, w1p, w2p)


_NBUF = 6
_RCH = 8
_LEAD = 3


def _dispatch(xs, src_flat):
    info = plsc.get_sparse_core_info()
    nc, ns = info.num_cores, info.num_subcores
    nw = nc * ns
    b = _E * _C
    bpw = b // nw
    r = _RCH
    nch = bpw // r

    sl = _D // 128

    @functools.partial(
        pl.kernel,
        mesh=plsc.VectorSubcoreMesh(core_axis_name="c", subcore_axis_name="s"),
        compiler_params=pltpu.CompilerParams(use_tc_tiling_on_sc=True),
        out_type=jax.ShapeDtypeStruct((b, sl, 128), jnp.float32),
        scratch_types=(
            [pltpu.VMEM((bpw,), jnp.int32)]
            + [pltpu.VMEM((r, sl, 128), jnp.float32)] * _NBUF
            + [pltpu.SemaphoreType.DMA] * (2 * _NBUF)
        ),
    )
    def k(xs_hbm, src_hbm, out_hbm, idx_v, *bufsem):
        bufs = bufsem[:_NBUF]
        gsem = bufsem[_NBUF:2 * _NBUF]
        ssem = bufsem[2 * _NBUF:]
        wid = lax.axis_index("s") * nc + lax.axis_index("c")
        base = wid * bpw
        pltpu.sync_copy(src_hbm.at[pl.ds(base, bpw)], idx_v)

        def start_gather(g, bi):
            pltpu.async_copy(
                xs_hbm.at[idx_v.at[pl.ds(g * r, r)]], bufs[bi], gsem[bi])

        def wait_dma(bi, sem):
            pltpu.make_async_copy(
                xs_hbm.at[pl.ds(0, r)], bufs[bi], sem[bi]).wait()

        def start_store(g, bi):
            pltpu.async_copy(
                bufs[bi], out_hbm.at[pl.ds(base + g * r, r)], ssem[bi])

        for g in range(_LEAD):
            start_gather(g, g)

        def step(kk, carry):
            for bi in range(_NBUF):
                g = _NBUF * kk + bi

                @pl.when(g < nch)
                def _(g=g, bi=bi):
                    wait_dma(bi, gsem)
                    start_store(g, bi)
                    bn = (bi + _LEAD) % _NBUF

                    @pl.when(g + _LEAD < nch)
                    def _():
                        @pl.when(g >= _NBUF - _LEAD)
                        def _():
                            wait_dma(bn, ssem)
                        start_gather(g + _LEAD, bn)

            return carry

        lax.fori_loop(0, (nch + _NBUF - 1) // _NBUF, step, 0)
        for g in range(nch - _NBUF, nch):
            wait_dma(g % _NBUF, ssem)

    return k(xs, src_flat)


def kernel(x, router_logits):
    src3, w1p, w2p = _routing(router_logits)
    xs = _tables(x, w1p, w2p)
    out = _dispatch(xs, src3.reshape(_E * _C))
    return out.reshape(_E, _C, _D)


# speedup vs baseline: 1.9836x; 1.8299x over previous
import functools

import jax
import jax.numpy as jnp
from jax import lax
from jax.experimental import pallas as pl
from jax.experimental.pallas import tpu as pltpu
from jax.experimental.pallas import tpu_sc as plsc

_T = 4096
_D = 2048
_E = 64
_C = 256
_L = 16
_TB = 512
_ZROW = 2 * _T


def _routing_body(logits_ref, src_ref, w1_ref, w2_ref, probs_ref, dstx_ref):
    e = pl.program_id(0)

    @pl.when(e == 0)
    def _init():
        l = logits_ref[...]
        m = jnp.max(l, axis=1, keepdims=True)
        p = jnp.exp(l - m)
        probs = p / jnp.sum(p, axis=1, keepdims=True)
        probs_ref[...] = probs
        cols = lax.broadcasted_iota(jnp.int32, (_T, _E), 1)
        m1 = jnp.max(probs, axis=1, keepdims=True)
        i1 = jnp.min(jnp.where(probs == m1, cols, _E), axis=1, keepdims=True)
        p2 = jnp.where(cols == i1, -jnp.inf, probs)
        m2 = jnp.max(p2, axis=1, keepdims=True)
        i2 = jnp.min(jnp.where(p2 == m2, cols, _E), axis=1, keepdims=True)
        is2 = (cols == i2).astype(jnp.int32)
        mask = (cols == i1).astype(jnp.int32) + is2
        c = mask
        k = 1
        while k < _T:
            c = c + jnp.concatenate(
                [jnp.zeros((k, _E), jnp.int32), c[:-k]], axis=0)
            k *= 2
        dstx_ref[...] = c * mask + 16384 * is2
        w1_ref[...] = jnp.broadcast_to(m1, (_T, 128))
        w2_ref[...] = jnp.broadcast_to(m2, (_T, 128))

    e_cols = lax.broadcasted_iota(jnp.int32, (_T, _E), 1)
    sel = e_cols == e
    dstx_col = jnp.sum(jnp.where(sel, dstx_ref[...], 0), axis=1)
    c2 = dstx_col // 16384
    dval = dstx_col - c2 * 16384
    slot = lax.broadcasted_iota(jnp.int32, (_T, _C), 1) + 1
    eq = slot == dval[:, None]
    ids = lax.broadcasted_iota(jnp.int32, (_T, _C), 0)
    idsx = ids + _T * c2[:, None]
    src_row = jnp.sum(jnp.where(eq, idsx - _ZROW, 0), axis=0) + _ZROW
    src_ref[...] = src_row.reshape(1, 1, _C)


def _routing(logits):
    return pl.pallas_call(
        _routing_body,
        grid=(_E,),
        in_specs=[pl.BlockSpec((_T, _E), lambda e: (0, 0))],
        out_specs=[
            pl.BlockSpec((1, 1, _C), lambda e: (e, 0, 0)),
            pl.BlockSpec((_T, 128), lambda e: (0, 0)),
            pl.BlockSpec((_T, 128), lambda e: (0, 0)),
        ],
        out_shape=[
            jax.ShapeDtypeStruct((_E, 1, _C), jnp.int32),
            jax.ShapeDtypeStruct((_T, 128), jnp.float32),
            jax.ShapeDtypeStruct((_T, 128), jnp.float32),
        ],
        scratch_shapes=[
            pltpu.VMEM((_T, _E), jnp.float32),
            pltpu.VMEM((_T, _E), jnp.int32),
        ],
    )(logits)


def _tables_body(x_ref, w1_ref, w2_ref, out_ref):
    s = pl.program_id(0)
    xb = x_ref[...]
    w1c = w1_ref[...][:, 0:1]
    w2c = w2_ref[...][:, 0:1]
    wc = jnp.where(s < 8, w1c, w2c)
    wc = jnp.where(s < 16, wc, 0.0)
    out_ref[...] = (xb * wc).reshape(_TB, _D // 128, 128)


def _tables(x, w1p, w2p):
    nblk = _T // _TB
    nsteps = 2 * nblk + 1
    return pl.pallas_call(
        _tables_body,
        grid=(nsteps,),
        in_specs=[
            pl.BlockSpec((_TB, _D), lambda s: (s % 8, 0)),
            pl.BlockSpec((_TB, 128), lambda s: (s % 8, 0)),
            pl.BlockSpec((_TB, 128), lambda s: (s % 8, 0)),
        ],
        out_specs=pl.BlockSpec((_TB, _D // 128, 128), lambda s: (s, 0, 0)),
        out_shape=jax.ShapeDtypeStruct(
            (nsteps * _TB, _D // 128, 128), jnp.float32),
    )(x, w1p, w2p)


_NBUF = 6
_RCH = 8
_LEAD = 3


def _dispatch(xs, src_flat):
    info = plsc.get_sparse_core_info()
    nc, ns = info.num_cores, info.num_subcores
    nw = nc * ns
    b = _E * _C
    bpw = b // nw
    r = _RCH
    nch = bpw // r

    sl = _D // 128

    @functools.partial(
        pl.kernel,
        mesh=plsc.VectorSubcoreMesh(core_axis_name="c", subcore_axis_name="s"),
        compiler_params=pltpu.CompilerParams(use_tc_tiling_on_sc=True),
        out_type=jax.ShapeDtypeStruct((b, sl, 128), jnp.float32),
        scratch_types=(
            [pltpu.VMEM((bpw + _L,), jnp.int32)]
            + [pltpu.VMEM((r, sl, 128), jnp.float32)] * (_NBUF + 1)
            + [pltpu.SemaphoreType.DMA] * (2 * _NBUF)
        ),
    )
    def k(xs_hbm, src_hbm, out_hbm, idx_v, *bufsem):
        bufs = bufsem[:_NBUF]
        zbuf = bufsem[_NBUF]
        gsem = bufsem[_NBUF + 1:2 * _NBUF + 1]
        ssem = bufsem[2 * _NBUF + 1:]
        wid = lax.axis_index("s") * nc + lax.axis_index("c")
        base = wid * bpw
        pltpu.sync_copy(src_hbm.at[pl.ds(base, bpw)], idx_v.at[pl.ds(0, bpw)])
        pltpu.sync_copy(xs_hbm.at[pl.ds(_ZROW, r)], zbuf)

        def is_live(g):
            return idx_v[pl.ds(g * r, _L)][0] != _ZROW

        def start_gather(g, bi):
            @pl.when(is_live(g))
            def _():
                pltpu.async_copy(
                    xs_hbm.at[idx_v.at[pl.ds(g * r, r)]], bufs[bi], gsem[bi])

        def wait_dma(bi, sem):
            pltpu.make_async_copy(
                xs_hbm.at[pl.ds(0, r)], bufs[bi], sem[bi]).wait()

        def start_store(g, bi):
            live = is_live(g)

            @pl.when(live)
            def _():
                pltpu.async_copy(
                    bufs[bi], out_hbm.at[pl.ds(base + g * r, r)], ssem[bi])

            @pl.when(jnp.logical_not(live))
            def _():
                pltpu.async_copy(
                    zbuf, out_hbm.at[pl.ds(base + g * r, r)], ssem[bi])

        for g in range(_LEAD):
            start_gather(g, g)

        def step(kk, carry):
            for bi in range(_NBUF):
                g = _NBUF * kk + bi

                @pl.when(g < nch)
                def _(g=g, bi=bi):
                    @pl.when(is_live(g))
                    def _():
                        wait_dma(bi, gsem)
                    start_store(g, bi)
                    bn = (bi + _LEAD) % _NBUF

                    @pl.when(g + _LEAD < nch)
                    def _():
                        @pl.when(g >= _NBUF - _LEAD)
                        def _():
                            wait_dma(bn, ssem)
                        start_gather(g + _LEAD, bn)

            return carry

        lax.fori_loop(0, (nch + _NBUF - 1) // _NBUF, step, 0)
        for g in range(nch - _NBUF, nch):
            wait_dma(g % _NBUF, ssem)

    return k(xs, src_flat)


def kernel(x, router_logits):
    src3, w1p, w2p = _routing(router_logits)
    xs = _tables(x, w1p, w2p)
    out = _dispatch(xs, src3.reshape(_E * _C))
    return out.reshape(_E, _C, _D)
